# baseline (speedup 1.0000x reference)
import functools

import jax
import jax.numpy as jnp
from jax import lax
from jax.experimental import pallas as pl
from jax.experimental.pallas import tpu as pltpu
from jax.experimental.pallas import tpu_sc as plsc

N = 10000
E = 160000
IN_FEAT = 256
HID = 256
HC = 512
OUT = 40
BETA = 0.5
EPS = 1e-5

NPAD = 10016
E2 = 163840
EPW_A = E2 // 32
KA = 64
EPT_B = E2 // 16
KB = 128
RPT = NPAD // 16


def _ln(x, g, b):
    mu = x.mean(axis=-1, keepdims=True)
    var = ((x - mu) ** 2).mean(axis=-1, keepdims=True)
    return (x - mu) / jnp.sqrt(var + EPS) * g + b


def _dense0_body(x_ref, w_ref, b_ref, g_ref, bb_ref, o_ref):
    h = jnp.dot(x_ref[...], w_ref[...], preferred_element_type=jnp.float32)
    h = jax.nn.relu(h + b_ref[...])
    o_ref[...] = _ln(h, g_ref[...], bb_ref[...])


def _dense0(x, W1, b1, g, bb):
    BM = 1000
    return pl.pallas_call(
        _dense0_body,
        grid=(N // BM,),
        in_specs=[
            pl.BlockSpec((BM, IN_FEAT), lambda i: (i, 0)),
            pl.BlockSpec((IN_FEAT, HC), lambda i: (0, 0)),
            pl.BlockSpec((HC,), lambda i: (0,)),
            pl.BlockSpec((HC,), lambda i: (0,)),
            pl.BlockSpec((HC,), lambda i: (0,)),
        ],
        out_specs=pl.BlockSpec((BM, HC), lambda i: (i, 0)),
        out_shape=jax.ShapeDtypeStruct((N, HC), jnp.float32),
    )(x, W1, b1, g, bb)


def _mm_body(a_ref, w_ref, o_ref, oh_ref, o4_ref):
    o = jnp.dot(a_ref[...], w_ref[...], preferred_element_type=jnp.float32)
    o_ref[...] = o
    oh_ref[...] = 0.5 * o
    for g in range(4):
        o4_ref[g] = o[:, g * 64:(g + 1) * 64]


def _mm(h, w):
    BM = 1000
    return pl.pallas_call(
        _mm_body,
        grid=(N // BM,),
        in_specs=[
            pl.BlockSpec((BM, HC), lambda i: (i, 0)),
            pl.BlockSpec((HC, HID), lambda i: (0, 0)),
        ],
        out_specs=[
            pl.BlockSpec((BM, HID), lambda i: (i, 0)),
            pl.BlockSpec((BM, HID), lambda i: (i, 0)),
            pl.BlockSpec((4, BM, 64), lambda i: (0, i, 0)),
        ],
        out_shape=[
            jax.ShapeDtypeStruct((N, HID), jnp.float32),
            jax.ShapeDtypeStruct((N, HID), jnp.float32),
            jax.ShapeDtypeStruct((4, N, 64), jnp.float32),
        ],
    )(h, w)


def _post_body(s0_ref, sa_ref, ego_ref, g_ref, b_ref, o_ref):
    s0 = s0_ref[...]
    seg = jnp.concatenate([s0, sa_ref[...] - s0], axis=-1)
    h2 = _ln(jax.nn.relu(seg), g_ref[...], b_ref[...])
    o_ref[...] = (1.0 - BETA) * h2 + BETA * ego_ref[...]


def _post(seg0, segA, ego, g, b):
    BM = 1000
    return pl.pallas_call(
        _post_body,
        grid=(N // BM,),
        in_specs=[
            pl.BlockSpec((BM, HID), lambda i: (i, 0)),
            pl.BlockSpec((BM, HID), lambda i: (i, 0)),
            pl.BlockSpec((BM, HC), lambda i: (i, 0)),
            pl.BlockSpec((HC,), lambda i: (0,)),
            pl.BlockSpec((HC,), lambda i: (0,)),
        ],
        out_specs=pl.BlockSpec((BM, HC), lambda i: (i, 0)),
        out_shape=jax.ShapeDtypeStruct((N, HC), jnp.float32),
    )(seg0, segA, ego, g, b)


def _final_body(h_ref, w_ref, b_ref, o_ref):
    o = jnp.dot(h_ref[...], w_ref[...], preferred_element_type=jnp.float32) + b_ref[...]
    o_ref[...] = jax.nn.log_softmax(o, axis=-1)


def _final(h, W2, b2):
    BM = 1000
    return pl.pallas_call(
        _final_body,
        grid=(N // BM,),
        in_specs=[
            pl.BlockSpec((BM, HC), lambda i: (i, 0)),
            pl.BlockSpec((HC, OUT), lambda i: (0, 0)),
            pl.BlockSpec((OUT,), lambda i: (0,)),
        ],
        out_specs=pl.BlockSpec((BM, OUT), lambda i: (i, 0)),
        out_shape=jax.ShapeDtypeStruct((N, OUT), jnp.float32),
    )(h, W2, b2)


def _att_body(hph_hbm, hp_hbm, row_hbm, col_hbm, wab_hbm, att_hbm,
              rbufs, cbufs, ridxs, cidxs, abufs, wavb,
              sem_gr, sem_gc, sem_i, sem_o):
    c = lax.axis_index("c")
    s = lax.axis_index("s")
    wid = s * 2 + c
    pltpu.sync_copy(wab_hbm, wavb)
    ev0 = lax.iota(jnp.int32, 16)
    evs = [ev0 + 16 * g for g in range(KA // 16)]
    base0 = wid * EPW_A

    def issue_idx(ci, b):
        base = base0 + ci * KA
        pltpu.async_copy(row_hbm.at[pl.ds(base, KA)], ridxs[b], sem_i[b])
        pltpu.async_copy(col_hbm.at[pl.ds(base, KA)], cidxs[b], sem_i[b])

    def wait_idx(b):
        pltpu.make_async_copy(row_hbm.at[pl.ds(0, KA)], ridxs[b], sem_i[b]).wait()
        pltpu.make_async_copy(col_hbm.at[pl.ds(0, KA)], cidxs[b], sem_i[b]).wait()

    def issue_gather(b):
        pltpu.async_copy(hph_hbm.at[ridxs[b]], rbufs[b], sem_gr[b])
        pltpu.async_copy(hp_hbm.at[cidxs[b]], cbufs[b], sem_gc[b])

    def wait_gather(b):
        pltpu.make_async_copy(hph_hbm.at[ridxs[b]], rbufs[b], sem_gr[b]).wait()
        pltpu.make_async_copy(hp_hbm.at[cidxs[b]], cbufs[b], sem_gc[b]).wait()

    issue_idx(0, 0)
    wait_idx(0)
    issue_gather(0)

    def outer(io, carry):
        for b in range(2):
            ci = io * 2 + b
            wait_gather(b)

            @pl.when(ci + 1 < EPW_A // KA)
            def _():
                issue_idx(ci + 1, 1 - b)

            rbuf, cbuf, abuf = rbufs[b], cbufs[b], abufs[b]

            @pl.when(ci >= 2)
            def _():
                pltpu.make_async_copy(
                    abuf, att_hbm.at[pl.ds(0, KA)], sem_o[b]).wait()

            ng = KA // 16

            def dot_body(jb, pc):
                jb16 = jb * 16
                pc = list(pc)
                for k in range(16):
                    d = jb16 + k
                    w0 = wavb[0, d]
                    w1 = wavb[1, d]
                    dv = jnp.full((16,), 0, dtype=jnp.int32) + d
                    for g in range(ng):
                        rv = plsc.load_gather(rbuf, [evs[g], dv])
                        cv = plsc.load_gather(cbuf, [evs[g], dv])
                        t = jnp.maximum(rv + cv, 0.0)
                        pc[2 * g] = pc[2 * g] + t * w0
                        pc[2 * g + 1] = pc[2 * g + 1] + t * w1
                return tuple(pc)

            z = jnp.zeros((16,), jnp.float32)
            pc = lax.fori_loop(0, HID // 16, dot_body, (z,) * (2 * ng))
            for g in range(ng):
                a = 1.0 / (1.0 + jnp.exp(pc[2 * g + 1] - pc[2 * g]))
                abuf[pl.ds(g * 16, 16)] = a
            base = base0 + ci * KA
            pltpu.async_copy(abuf, att_hbm.at[pl.ds(base, KA)], sem_o[b])

            @pl.when(ci + 1 < EPW_A // KA)
            def _():
                wait_idx(1 - b)
                issue_gather(1 - b)
        return carry

    lax.fori_loop(0, EPW_A // (2 * KA), outer, 0)
    for b in range(2):
        pltpu.make_async_copy(abufs[b], att_hbm.at[pl.ds(0, KA)], sem_o[b]).wait()


def _phase_a(hph_pad, hp_pad, row_p, col_p, waB):
    mesh = plsc.VectorSubcoreMesh(core_axis_name="c", subcore_axis_name="s")
    f = functools.partial(
        pl.kernel,
        out_type=jax.ShapeDtypeStruct((E2,), jnp.float32),
        mesh=mesh,
        compiler_params=pltpu.CompilerParams(use_tc_tiling_on_sc=False, needs_layout_passes=False),
        scratch_types=[
            [pltpu.VMEM((KA, HID), jnp.float32)] * 2,
            [pltpu.VMEM((KA, HID), jnp.float32)] * 2,
            [pltpu.VMEM((KA,), jnp.int32)] * 2,
            [pltpu.VMEM((KA,), jnp.int32)] * 2,
            [pltpu.VMEM((KA,), jnp.float32)] * 2,
            pltpu.VMEM((2, HID, 16), jnp.float32),
            [pltpu.SemaphoreType.DMA] * 2,
            [pltpu.SemaphoreType.DMA] * 2,
            [pltpu.SemaphoreType.DMA] * 2,
            [pltpu.SemaphoreType.DMA] * 2,
        ],
    )(_att_body)
    return f(hph_pad, hp_pad, row_p, col_p, waB)


def _agg_body(hp4f_hbm, row_hbm, col_hbm, att_hbm, zeros_hbm,
              out0_hbm, outA_hbm,
              acc0, accA, cbufs, sbufs, ridxs, cidxs, cidx2s, abufs,
              sem_g, sem_s0, sem_sA, sem_i):
    c = lax.axis_index("c")
    s = lax.axis_index("s")
    rows0 = s * RPT
    ev0 = lax.iota(jnp.int32, 16)
    evs = [ev0 + 16 * q for q in range(KB // 16)]
    nq = KB // 16
    NC = EPT_B // KB

    for lg in range(2):
        g = c * 2 + lg
        goff = g * NPAD
        pltpu.sync_copy(zeros_hbm.at[pl.ds(rows0, RPT)], acc0.at[pl.ds(rows0, RPT)])
        pltpu.sync_copy(zeros_hbm.at[pl.ds(rows0, RPT)], accA.at[pl.ds(rows0, RPT)])
        plsc.subcore_barrier()

        def issue_idx(ci, b):
            base = s * EPT_B + ci * KB
            pltpu.async_copy(row_hbm.at[pl.ds(base, KB)], ridxs[b], sem_i[b])
            pltpu.async_copy(col_hbm.at[pl.ds(base, KB)], cidxs[b], sem_i[b])
            pltpu.async_copy(att_hbm.at[pl.ds(base, KB)], abufs[b], sem_i[b])

        def wait_idx(b):
            pltpu.make_async_copy(row_hbm.at[pl.ds(0, KB)], ridxs[b], sem_i[b]).wait()
            pltpu.make_async_copy(col_hbm.at[pl.ds(0, KB)], cidxs[b], sem_i[b]).wait()
            pltpu.make_async_copy(att_hbm.at[pl.ds(0, KB)], abufs[b], sem_i[b]).wait()

        issue_idx(0, 0)
        wait_idx(0)
        for q in range(nq):
            cidx2s[0][pl.ds(q * 16, 16)] = cidxs[0][pl.ds(q * 16, 16)] + goff
        pltpu.async_copy(hp4f_hbm.at[cidx2s[0]], cbufs[0], sem_g[0])

        def outer(io, carry):
            for b in range(2):
                ci = io * 2 + b
                cbuf, sbuf, abuf = cbufs[b], sbufs[b], abufs[b]
                pltpu.make_async_copy(
                    hp4f_hbm.at[cidx2s[b]], cbuf, sem_g[b]).wait()

                @pl.when(ci + 1 < NC)
                def _():
                    issue_idx(ci + 1, 1 - b)

                @pl.when(ci >= 2)
                def _():
                    pltpu.make_async_copy(
                        sbuf, acc0.at[pl.ds(0, KB)], sem_s0[b]).wait()
                    pltpu.make_async_copy(
                        cbuf, accA.at[pl.ds(0, KB)], sem_sA[b]).wait()

                avs = [abuf[pl.ds(q * 16, 16)] for q in range(nq)]

                def d_body(d, carry2):
                    dv = jnp.full((16,), 0, dtype=jnp.int32) + d
                    for q in range(nq):
                        v = plsc.load_gather(cbuf, [evs[q], dv])
                        plsc.store_scatter(sbuf, [evs[q], dv], v * avs[q])
                    return carry2

                lax.fori_loop(0, 64, d_body, 0)
                cpA = pltpu.async_copy(cbuf, accA.at[ridxs[b]], sem_sA[b], add=True)
                cp0 = pltpu.async_copy(sbuf, acc0.at[ridxs[b]], sem_s0[b], add=True)

                @pl.when(ci + 1 < NC)
                def _():
                    wait_idx(1 - b)
                    for q in range(nq):
                        cidx2s[1 - b][pl.ds(q * 16, 16)] = (
                            cidxs[1 - b][pl.ds(q * 16, 16)] + goff)
                    pltpu.async_copy(hp4f_hbm.at[cidx2s[1 - b]], cbufs[1 - b],
                                     sem_g[1 - b])
            return carry

        lax.fori_loop(0, NC // 2, outer, 0)
        for b in range(2):
            pltpu.make_async_copy(sbufs[b], acc0.at[pl.ds(0, KB)], sem_s0[b]).wait()
            pltpu.make_async_copy(cbufs[b], accA.at[pl.ds(0, KB)], sem_sA[b]).wait()
        plsc.subcore_barrier()
        pltpu.sync_copy(acc0.at[pl.ds(rows0, RPT)],
                        out0_hbm.at[pl.ds(goff + rows0, RPT)])
        pltpu.sync_copy(accA.at[pl.ds(rows0, RPT)],
                        outA_hbm.at[pl.ds(goff + rows0, RPT)])
        plsc.subcore_barrier()


def _phase_b(hp4f, row_p, col_p, att, zeros_rows):
    mesh = plsc.VectorSubcoreMesh(core_axis_name="c", subcore_axis_name="s")
    f = functools.partial(
        pl.kernel,
        out_type=[
            jax.ShapeDtypeStruct((4 * NPAD, 64), jnp.float32),
            jax.ShapeDtypeStruct((4 * NPAD, 64), jnp.float32),
        ],
        mesh=mesh,
        compiler_params=pltpu.CompilerParams(use_tc_tiling_on_sc=False, needs_layout_passes=False),
        scratch_types=[
            pltpu.VMEM_SHARED((NPAD, 64), jnp.float32),
            pltpu.VMEM_SHARED((NPAD, 64), jnp.float32),
            [pltpu.VMEM((KB, 64), jnp.float32)] * 2,
            [pltpu.VMEM((KB, 64), jnp.float32)] * 2,
            [pltpu.VMEM((KB,), jnp.int32)] * 2,
            [pltpu.VMEM((KB,), jnp.int32)] * 2,
            [pltpu.VMEM((KB,), jnp.int32)] * 2,
            [pltpu.VMEM((KB,), jnp.float32)] * 2,
            [pltpu.SemaphoreType.DMA] * 2,
            [pltpu.SemaphoreType.DMA] * 2,
            [pltpu.SemaphoreType.DMA] * 2,
            [pltpu.SemaphoreType.DMA] * 2,
        ],
    )(_agg_body)
    return f(hp4f, row_p, col_p, att, zeros_rows)


def _edge_layer(h, row_p, col_p, waB, w_lin, zeros_rows):
    hp, hph, hp4 = _mm(h, w_lin)
    hp_pad = jnp.pad(hp, ((0, NPAD - N), (0, 0)))
    hph_pad = jnp.pad(hph, ((0, NPAD - N), (0, 0)))
    hp4f = jnp.pad(hp4, ((0, 0), (0, NPAD - N), (0, 0))).reshape(4 * NPAD, 64)
    att = _phase_a(hph_pad, hp_pad, row_p, col_p, waB)
    out0f, outAf = _phase_b(hp4f, row_p, col_p, att, zeros_rows)
    seg0 = out0f.reshape(4, NPAD, 64)[:, :N, :].transpose(1, 0, 2).reshape(N, HID)
    segA = outAf.reshape(4, NPAD, 64)[:, :N, :].transpose(1, 0, 2).reshape(N, HID)
    return seg0, segA


def kernel(x, edge_index, W1, b1, ln0_g, ln0_b, lin_w0, att_w0, ln1_g, ln1_b,
           lin_w1, att_w1, ln2_g, ln2_b, W2, b2):
    row = edge_index[0]
    col = edge_index[1]
    row = jnp.where(row != col, row, N)
    row_p = jnp.concatenate([row, jnp.full((E2 - E,), N, jnp.int32)])
    col_p = jnp.concatenate([col, jnp.zeros((E2 - E,), jnp.int32)])
    zeros_rows = jnp.zeros((NPAD, 64), jnp.float32)

    h = _dense0(x, W1, b1, ln0_g, ln0_b)
    ego = h
    for (wl, wa, g, bb) in ((lin_w0, att_w0, ln1_g, ln1_b),
                            (lin_w1, att_w1, ln2_g, ln2_b)):
        waB = jnp.broadcast_to(wa.T[:, :, None], (2, HID, 16)) + 0.0
        seg0, segA = _edge_layer(h, row_p, col_p, waB, wl, zeros_rows)
        h = _post(seg0, segA, ego, g, bb)
    return _final(h, W2, b2)
